# Initial kernel scaffold; baseline (speedup 1.0000x reference)
import jax
import jax.numpy as jnp
from jax.experimental import pallas as pl


def kernel(x, weight, neg_t, emb_E_real, emb_E_img, emb_R_real, emb_R_img, emb_Time):
    raise NotImplementedError("write your pallas kernel here")


# speedup vs baseline: 10.7273x; 10.7273x over previous
import functools

import jax
import jax.numpy as jnp
from jax import lax
from jax.experimental import pallas as pl
from jax.experimental.pallas import tpu as pltpu
from jax.experimental.pallas import tpu_sc as plsc

DIM = 128
CAND = 50
CPAD = 64
NC, NS = 2, 16
NW = NC * NS
L = 16
NKK = DIM // L
CH = 32


def _trig_body(t_ref, c_ref, s_ref):
    v = t_ref[...]
    c_ref[...] = jnp.cos(v)
    s_ref[...] = jnp.sin(v)


def _time_trig(emb_time8):
    return pl.pallas_call(
        _trig_body,
        out_shape=(
            jax.ShapeDtypeStruct((8, DIM), jnp.float32),
            jax.ShapeDtypeStruct((8, DIM), jnp.float32),
        ),
    )(emb_time8)


def _sc_score(ct8, st8, h_idx, r_idx, d_idx, t_idx,
              emb_er, emb_ei, emb_rr, emb_ri):
    batch = h_idx.shape[0]
    rows = batch // NW
    mesh = plsc.VectorSubcoreMesh(core_axis_name="c", subcore_axis_name="s")

    @functools.partial(
        pl.kernel,
        out_type=jax.ShapeDtypeStruct((batch, CPAD + L), jnp.float32),
        mesh=mesh,
        scratch_types=[
            pltpu.VMEM((8, DIM), jnp.float32),
            pltpu.VMEM((8, DIM), jnp.float32),
            pltpu.VMEM((rows,), jnp.int32),
            pltpu.VMEM((rows,), jnp.int32),
            pltpu.VMEM((rows, L), jnp.int32),
            pltpu.VMEM((rows, CAND), jnp.int32),
            pltpu.VMEM((CH, DIM), jnp.float32),
            pltpu.VMEM((CH, DIM), jnp.float32),
            pltpu.VMEM((CH, DIM), jnp.float32),
            pltpu.VMEM((CH, DIM), jnp.float32),
            pltpu.VMEM((CAND, DIM), jnp.float32),
            pltpu.VMEM((CAND, DIM), jnp.float32),
            pltpu.VMEM((rows, CPAD + L), jnp.float32),
            pltpu.VMEM((2 * L,), jnp.float32),
            pltpu.SemaphoreType.DMA,
            pltpu.SemaphoreType.DMA,
        ],
    )
    def k(ct_h, st_h, hidx_h, ridx_h, didx_h, tidx_h, er_h, ei_h, rr_h, ri_h,
          out_h, ct_v, st_v, hidx_v, ridx_v, didx_v, tidx_v,
          her_v, hei_v, relr_v, reli_v, ter_v, tei_v, out_v, fold_v,
          sem0, sem1):
        wid = lax.axis_index("s") * NC + lax.axis_index("c")
        base = wid * rows

        pltpu.sync_copy(ct_h, ct_v)
        pltpu.sync_copy(st_h, st_v)
        pltpu.sync_copy(hidx_h.at[pl.ds(base, rows)], hidx_v)
        pltpu.sync_copy(ridx_h.at[pl.ds(base, rows)], ridx_v)
        pltpu.sync_copy(didx_h.at[pl.ds(base, rows)], didx_v)
        pltpu.sync_copy(tidx_h.at[pl.ds(base, rows)], tidx_v)

        lane = lax.iota(jnp.int32, L)
        fold_v[pl.ds(L, L)] = jnp.zeros((L,), jnp.float32)

        def chunk_body(ci, carry0):
            rb = ci * CH
            sl_rows = pl.ds(rb, CH)
            pltpu.async_copy(er_h.at[hidx_v.at[sl_rows]], her_v, sem0).wait()
            pltpu.async_copy(ei_h.at[hidx_v.at[sl_rows]], hei_v, sem0).wait()
            pltpu.async_copy(rr_h.at[ridx_v.at[sl_rows]], relr_v, sem0).wait()
            pltpu.async_copy(ri_h.at[ridx_v.at[sl_rows]], reli_v, sem0).wait()

            def row_body(bb, carry):
                b = rb + bb
                pltpu.async_copy(er_h.at[tidx_v.at[b]], ter_v, sem0).wait()
                pltpu.async_copy(ei_h.at[tidx_v.at[b]], tei_v, sem1).wait()

                dvec = didx_v[b, :]
                m0 = dvec == 0
                m1 = dvec == 1
                ccs, sss, aas, bbs = [], [], [], []
                for kk in range(NKK):
                    sl = pl.ds(kk * L, L)
                    cc = jnp.where(m0, ct_v[0, sl],
                                   jnp.where(m1, ct_v[1, sl], ct_v[2, sl]))
                    ss = jnp.where(m0, st_v[0, sl],
                                   jnp.where(m1, st_v[1, sl], st_v[2, sl]))
                    hr = her_v[bb, sl]
                    hi = hei_v[bb, sl]
                    a = hr * cc - hi * ss + relr_v[bb, sl]
                    bb_ = hr * ss + hi * cc + reli_v[bb, sl]
                    ccs.append(cc)
                    sss.append(ss)
                    aas.append(a)
                    bbs.append(bb_)

                def jbody(j, c):
                    acc = jnp.zeros((L,), jnp.float32)
                    for kk in range(NKK):
                        sl = pl.ds(kk * L, L)
                        tr = ter_v[j, sl]
                        ti = tei_v[j, sl]
                        acc = (acc
                               + jnp.abs(aas[kk] - (tr * ccs[kk] - ti * sss[kk]))
                               + jnp.abs(bbs[kk] + (tr * sss[kk] + ti * ccs[kk])))
                    for sh in (8, 4, 2, 1):
                        fold_v[pl.ds(0, L)] = acc
                        acc = acc + fold_v[pl.ds(sh, L)]
                    accz = jnp.where(lane == 0, acc, 0.0)
                    out_v[b, pl.ds(j, L)] = accz
                    return c

                lax.fori_loop(0, CAND, jbody, 0, unroll=2)
                return carry

            lax.fori_loop(0, CH, row_body, 0)
            return carry0

        lax.fori_loop(0, rows // CH, chunk_body, 0)
        pltpu.sync_copy(out_v, out_h.at[pl.ds(base, rows)])

    return k(ct8, st8, h_idx, r_idx, d_idx, t_idx,
             emb_er, emb_ei, emb_rr, emb_ri)


def kernel(x, weight, neg_t, emb_E_real, emb_E_img, emb_R_real, emb_R_img,
           emb_Time):
    del weight
    h_idx = x[:, 0]
    r_idx = x[:, 1]
    d_idx = jnp.tile((x[:, 3] % 3)[:, None], (1, 16))
    t_idx = jnp.concatenate([x[:, 2:3], neg_t], axis=1)
    ct8, st8 = _time_trig(emb_Time[:8])
    out = _sc_score(ct8, st8, h_idx, r_idx, d_idx, t_idx,
                    emb_E_real, emb_E_img, emb_R_real, emb_R_img)
    return out[:, :CAND]


# speedup vs baseline: 17.6647x; 1.6467x over previous
import functools

import jax
import jax.numpy as jnp
from jax import lax
from jax.experimental import pallas as pl
from jax.experimental.pallas import tpu as pltpu
from jax.experimental.pallas import tpu_sc as plsc

DIM = 128
CAND = 50
CPAD = 64
NC, NS = 2, 16
NW = NC * NS
L = 16
NKK = DIM // L
CH = 32


def _trig_body(t_ref, c_ref, s_ref):
    v = t_ref[...]
    c_ref[...] = jnp.cos(v)
    s_ref[...] = jnp.sin(v)


def _time_trig(emb_time8):
    return pl.pallas_call(
        _trig_body,
        out_shape=(
            jax.ShapeDtypeStruct((8, DIM), jnp.float32),
            jax.ShapeDtypeStruct((8, DIM), jnp.float32),
        ),
    )(emb_time8)


def _sc_score(ct8, st8, h_idx, r_idx, d_idx, t_idx,
              emb_er, emb_ei, emb_rr, emb_ri):
    batch = h_idx.shape[0]
    rows = batch // NW
    mesh = plsc.VectorSubcoreMesh(core_axis_name="c", subcore_axis_name="s")

    @functools.partial(
        pl.kernel,
        out_type=jax.ShapeDtypeStruct((batch, CPAD + L), jnp.float32),
        mesh=mesh,
        scratch_types=[
            pltpu.VMEM((8, DIM), jnp.float32),
            pltpu.VMEM((8, DIM), jnp.float32),
            pltpu.VMEM((rows,), jnp.int32),
            pltpu.VMEM((rows,), jnp.int32),
            pltpu.VMEM((rows, L), jnp.int32),
            pltpu.VMEM((rows, CAND), jnp.int32),
            pltpu.VMEM((CH, DIM), jnp.float32),
            pltpu.VMEM((CH, DIM), jnp.float32),
            pltpu.VMEM((CH, DIM), jnp.float32),
            pltpu.VMEM((CH, DIM), jnp.float32),
            pltpu.VMEM((CAND, DIM), jnp.float32),
            pltpu.VMEM((CAND, DIM), jnp.float32),
            pltpu.VMEM((CAND, DIM), jnp.float32),
            pltpu.VMEM((CAND, DIM), jnp.float32),
            pltpu.VMEM((rows, CPAD + L), jnp.float32),
            pltpu.VMEM((2 * L,), jnp.float32),
            pltpu.SemaphoreType.DMA,
            pltpu.SemaphoreType.DMA,
            pltpu.SemaphoreType.DMA,
            pltpu.SemaphoreType.DMA,
            pltpu.SemaphoreType.DMA,
        ],
    )
    def k(ct_h, st_h, hidx_h, ridx_h, didx_h, tidx_h, er_h, ei_h, rr_h, ri_h,
          out_h, ct_v, st_v, hidx_v, ridx_v, didx_v, tidx_v,
          her_v, hei_v, relr_v, reli_v, ter_a, tei_a, ter_b, tei_b,
          out_v, fold_v, sem0, sem_ar, sem_ai, sem_br, sem_bi):
        wid = lax.axis_index("s") * NC + lax.axis_index("c")
        base = wid * rows

        pltpu.sync_copy(ct_h, ct_v)
        pltpu.sync_copy(st_h, st_v)
        pltpu.sync_copy(hidx_h.at[pl.ds(base, rows)], hidx_v)
        pltpu.sync_copy(ridx_h.at[pl.ds(base, rows)], ridx_v)
        pltpu.sync_copy(didx_h.at[pl.ds(base, rows)], didx_v)
        pltpu.sync_copy(tidx_h.at[pl.ds(base, rows)], tidx_v)

        lane = lax.iota(jnp.int32, L)
        fold_v[pl.ds(L, L)] = jnp.zeros((L,), jnp.float32)

        def start_tails(b, ter, tei, semr, semi):
            pltpu.async_copy(er_h.at[tidx_v.at[b]], ter, semr)
            pltpu.async_copy(ei_h.at[tidx_v.at[b]], tei, semi)

        def wait_tails(b, ter, tei, semr, semi):
            pltpu.make_async_copy(er_h.at[tidx_v.at[b]], ter, semr).wait()
            pltpu.make_async_copy(ei_h.at[tidx_v.at[b]], tei, semi).wait()

        def compute_row(b, bb, ter_v, tei_v):
            dvec = didx_v[b, :]
            m0 = dvec == 0
            m1 = dvec == 1
            ccs, sss, aas, bbs = [], [], [], []
            for kk in range(NKK):
                sl = pl.ds(kk * L, L)
                cc = jnp.where(m0, ct_v[0, sl],
                               jnp.where(m1, ct_v[1, sl], ct_v[2, sl]))
                ss = jnp.where(m0, st_v[0, sl],
                               jnp.where(m1, st_v[1, sl], st_v[2, sl]))
                hr = her_v[bb, sl]
                hi = hei_v[bb, sl]
                a = hr * cc - hi * ss + relr_v[bb, sl]
                bb_ = hr * ss + hi * cc + reli_v[bb, sl]
                ccs.append(cc)
                sss.append(ss)
                aas.append(a)
                bbs.append(bb_)

            def jbody(j, c):
                acc = jnp.zeros((L,), jnp.float32)
                for kk in range(NKK):
                    sl = pl.ds(kk * L, L)
                    tr = ter_v[j, sl]
                    ti = tei_v[j, sl]
                    acc = (acc
                           + jnp.abs(aas[kk] - (tr * ccs[kk] - ti * sss[kk]))
                           + jnp.abs(bbs[kk] + (tr * sss[kk] + ti * ccs[kk])))
                for sh in (8, 4, 2, 1):
                    fold_v[pl.ds(0, L)] = acc
                    acc = acc + fold_v[pl.ds(sh, L)]
                accz = jnp.where(lane == 0, acc, 0.0)
                out_v[b, pl.ds(j, L)] = accz
                return c

            lax.fori_loop(0, CAND, jbody, 0, unroll=2)

        start_tails(0, ter_a, tei_a, sem_ar, sem_ai)

        def chunk_body(ci, carry0):
            rb = ci * CH
            sl_rows = pl.ds(rb, CH)
            pltpu.async_copy(er_h.at[hidx_v.at[sl_rows]], her_v, sem0).wait()
            pltpu.async_copy(ei_h.at[hidx_v.at[sl_rows]], hei_v, sem0).wait()
            pltpu.async_copy(rr_h.at[ridx_v.at[sl_rows]], relr_v, sem0).wait()
            pltpu.async_copy(ri_h.at[ridx_v.at[sl_rows]], reli_v, sem0).wait()

            def pair_body(i, carry):
                b0 = rb + 2 * i
                start_tails(b0 + 1, ter_b, tei_b, sem_br, sem_bi)
                wait_tails(b0, ter_a, tei_a, sem_ar, sem_ai)
                compute_row(b0, 2 * i, ter_a, tei_a)
                bn = jnp.minimum(b0 + 2, rows - 1)
                start_tails(bn, ter_a, tei_a, sem_ar, sem_ai)
                wait_tails(b0 + 1, ter_b, tei_b, sem_br, sem_bi)
                compute_row(b0 + 1, 2 * i + 1, ter_b, tei_b)
                return carry

            lax.fori_loop(0, CH // 2, pair_body, 0)
            return carry0

        lax.fori_loop(0, rows // CH, chunk_body, 0)
        wait_tails(rows - 1, ter_a, tei_a, sem_ar, sem_ai)
        pltpu.sync_copy(out_v, out_h.at[pl.ds(base, rows)])

    return k(ct8, st8, h_idx, r_idx, d_idx, t_idx,
             emb_er, emb_ei, emb_rr, emb_ri)


def kernel(x, weight, neg_t, emb_E_real, emb_E_img, emb_R_real, emb_R_img,
           emb_Time):
    del weight
    h_idx = x[:, 0]
    r_idx = x[:, 1]
    d_idx = jnp.tile((x[:, 3] % 3)[:, None], (1, 16))
    t_idx = jnp.concatenate([x[:, 2:3], neg_t], axis=1)
    ct8, st8 = _time_trig(emb_Time[:8])
    out = _sc_score(ct8, st8, h_idx, r_idx, d_idx, t_idx,
                    emb_E_real, emb_E_img, emb_R_real, emb_R_img)
    return out[:, :CAND]


# speedup vs baseline: 28.3137x; 1.6028x over previous
import functools

import jax
import jax.numpy as jnp
from jax import lax
from jax.experimental import pallas as pl
from jax.experimental.pallas import tpu as pltpu
from jax.experimental.pallas import tpu_sc as plsc

DIM = 128
CAND = 50
CPAD = 64
NC, NS = 2, 16
NW = NC * NS
L = 16
NKK = DIM // L
CH = 32
JU = 5


def _trig_body(t_ref, c_ref, s_ref):
    v = t_ref[...]
    c_ref[...] = jnp.cos(v)
    s_ref[...] = jnp.sin(v)


def _time_trig(emb_time8):
    return pl.pallas_call(
        _trig_body,
        out_shape=(
            jax.ShapeDtypeStruct((8, DIM), jnp.float32),
            jax.ShapeDtypeStruct((8, DIM), jnp.float32),
        ),
    )(emb_time8)


def _sc_score(ct8, st8, h_idx, r_idx, d_idx, t_idx,
              emb_er, emb_ei, emb_rr, emb_ri):
    batch = h_idx.shape[0]
    rows = batch // NW
    mesh = plsc.VectorSubcoreMesh(core_axis_name="c", subcore_axis_name="s")

    @functools.partial(
        pl.kernel,
        out_type=jax.ShapeDtypeStruct((batch, CPAD + L), jnp.float32),
        mesh=mesh,
        scratch_types=[
            pltpu.VMEM((8, DIM), jnp.float32),
            pltpu.VMEM((8, DIM), jnp.float32),
            pltpu.VMEM((rows,), jnp.int32),
            pltpu.VMEM((rows,), jnp.int32),
            pltpu.VMEM((rows, L), jnp.int32),
            pltpu.VMEM((rows, CAND), jnp.int32),
            pltpu.VMEM((CH, DIM), jnp.float32),
            pltpu.VMEM((CH, DIM), jnp.float32),
            pltpu.VMEM((CH, DIM), jnp.float32),
            pltpu.VMEM((CH, DIM), jnp.float32),
            pltpu.VMEM((CAND, DIM), jnp.float32),
            pltpu.VMEM((CAND, DIM), jnp.float32),
            pltpu.VMEM((CAND, DIM), jnp.float32),
            pltpu.VMEM((CAND, DIM), jnp.float32),
            pltpu.VMEM((rows, CPAD + L), jnp.float32),
            pltpu.VMEM((DIM,), jnp.float32),
            pltpu.VMEM((DIM,), jnp.float32),
            pltpu.VMEM((DIM,), jnp.float32),
            pltpu.VMEM((DIM,), jnp.float32),
            pltpu.VMEM((JU * 2 * L,), jnp.float32),
            pltpu.SemaphoreType.DMA,
            pltpu.SemaphoreType.DMA,
            pltpu.SemaphoreType.DMA,
            pltpu.SemaphoreType.DMA,
            pltpu.SemaphoreType.DMA,
        ],
    )
    def k(ct_h, st_h, hidx_h, ridx_h, didx_h, tidx_h, er_h, ei_h, rr_h, ri_h,
          out_h, ct_v, st_v, hidx_v, ridx_v, didx_v, tidx_v,
          her_v, hei_v, relr_v, reli_v, ter_a, tei_a, ter_b, tei_b,
          out_v, av_v, bv_v, cv_v, sv_v, fold_v,
          sem0, sem_ar, sem_ai, sem_br, sem_bi):
        wid = lax.axis_index("s") * NC + lax.axis_index("c")
        base = wid * rows

        pltpu.sync_copy(ct_h, ct_v)
        pltpu.sync_copy(st_h, st_v)
        pltpu.sync_copy(hidx_h.at[pl.ds(base, rows)], hidx_v)
        pltpu.sync_copy(ridx_h.at[pl.ds(base, rows)], ridx_v)
        pltpu.sync_copy(didx_h.at[pl.ds(base, rows)], didx_v)
        pltpu.sync_copy(tidx_h.at[pl.ds(base, rows)], tidx_v)

        lane = lax.iota(jnp.int32, L)
        for u in range(JU):
            fold_v[pl.ds(u * 2 * L + L, L)] = jnp.zeros((L,), jnp.float32)

        def start_tails(b, ter, tei, semr, semi):
            pltpu.async_copy(er_h.at[tidx_v.at[b]], ter, semr)
            pltpu.async_copy(ei_h.at[tidx_v.at[b]], tei, semi)

        def wait_tails(b, ter, tei, semr, semi):
            pltpu.make_async_copy(er_h.at[tidx_v.at[b]], ter, semr).wait()
            pltpu.make_async_copy(ei_h.at[tidx_v.at[b]], tei, semi).wait()

        def compute_row(b, bb, ter_v, tei_v):
            dvec = didx_v[b, :]
            m0 = dvec == 0
            m1 = dvec == 1
            for kk in range(NKK):
                sl = pl.ds(kk * L, L)
                cc = jnp.where(m0, ct_v[0, sl],
                               jnp.where(m1, ct_v[1, sl], ct_v[2, sl]))
                ss = jnp.where(m0, st_v[0, sl],
                               jnp.where(m1, st_v[1, sl], st_v[2, sl]))
                hr = her_v[bb, sl]
                hi = hei_v[bb, sl]
                av_v[sl] = hr * cc - hi * ss + relr_v[bb, sl]
                bv_v[sl] = hr * ss + hi * cc + reli_v[bb, sl]
                cv_v[sl] = cc
                sv_v[sl] = ss

            def jbody(jt, c):
                j0 = jt * JU
                accs = [None] * JU
                for kk in range(NKK):
                    sl = pl.ds(kk * L, L)
                    a = av_v[sl]
                    bb_ = bv_v[sl]
                    cc = cv_v[sl]
                    ss = sv_v[sl]
                    for u in range(JU):
                        j = j0 + u
                        tr = ter_v[j, sl]
                        ti = tei_v[j, sl]
                        t = (jnp.abs(a - (tr * cc - ti * ss))
                             + jnp.abs(bb_ + (tr * ss + ti * cc)))
                        accs[u] = t if kk == 0 else accs[u] + t
                for u in range(JU):
                    acc = accs[u]
                    fb = u * 2 * L
                    for sh in (8, 4, 2, 1):
                        fold_v[pl.ds(fb, L)] = acc
                        acc = acc + fold_v[pl.ds(fb + sh, L)]
                    accs[u] = acc
                for u in range(JU):
                    accz = jnp.where(lane == 0, accs[u], 0.0)
                    out_v[b, pl.ds(j0 + u, L)] = accz
                return c

            lax.fori_loop(0, CAND // JU, jbody, 0)

        start_tails(0, ter_a, tei_a, sem_ar, sem_ai)

        def chunk_body(ci, carry0):
            rb = ci * CH
            sl_rows = pl.ds(rb, CH)
            pltpu.async_copy(er_h.at[hidx_v.at[sl_rows]], her_v, sem0).wait()
            pltpu.async_copy(ei_h.at[hidx_v.at[sl_rows]], hei_v, sem0).wait()
            pltpu.async_copy(rr_h.at[ridx_v.at[sl_rows]], relr_v, sem0).wait()
            pltpu.async_copy(ri_h.at[ridx_v.at[sl_rows]], reli_v, sem0).wait()

            def pair_body(i, carry):
                b0 = rb + 2 * i
                start_tails(b0 + 1, ter_b, tei_b, sem_br, sem_bi)
                wait_tails(b0, ter_a, tei_a, sem_ar, sem_ai)
                compute_row(b0, 2 * i, ter_a, tei_a)
                bn = jnp.minimum(b0 + 2, rows - 1)
                start_tails(bn, ter_a, tei_a, sem_ar, sem_ai)
                wait_tails(b0 + 1, ter_b, tei_b, sem_br, sem_bi)
                compute_row(b0 + 1, 2 * i + 1, ter_b, tei_b)
                return carry

            lax.fori_loop(0, CH // 2, pair_body, 0)
            return carry0

        lax.fori_loop(0, rows // CH, chunk_body, 0)
        wait_tails(rows - 1, ter_a, tei_a, sem_ar, sem_ai)
        pltpu.sync_copy(out_v, out_h.at[pl.ds(base, rows)])

    return k(ct8, st8, h_idx, r_idx, d_idx, t_idx,
             emb_er, emb_ei, emb_rr, emb_ri)


def kernel(x, weight, neg_t, emb_E_real, emb_E_img, emb_R_real, emb_R_img,
           emb_Time):
    del weight
    h_idx = x[:, 0]
    r_idx = x[:, 1]
    d_idx = jnp.tile((x[:, 3] % 3)[:, None], (1, 16))
    t_idx = jnp.concatenate([x[:, 2:3], neg_t], axis=1)
    ct8, st8 = _time_trig(emb_Time[:8])
    out = _sc_score(ct8, st8, h_idx, r_idx, d_idx, t_idx,
                    emb_E_real, emb_E_img, emb_R_real, emb_R_img)
    return out[:, :CAND]
